# baseline (speedup 1.0000x reference)
import numpy as np
import jax
import jax.numpy as jnp
from jax import lax
from jax.experimental import pallas as pl

B = 4096
L = 200
CH = 25
G = L // CH
NL = CH * 20
R = B * G
PR = 1024
PB = PR // G
K1 = 5 * CH
K2 = K1 + 8 * CH
KS = K2 + 7 + 48 + 8


def _build_S(weekday_W, start_time_W, sem_pt_W, fc_W):
    rows, cols, kind = [], [], []
    k = np.arange(CH)
    for c in range(5):
        rows.append(c * CH + k)
        cols.append(20 * k + c)
        kind.append(np.full(CH, -1))
    for cat in range(1, 9):
        for j in range(3):
            rows.append(K1 + (cat - 1) * CH + k)
            cols.append(20 * k + 17 + j)
            kind.append(np.full(CH, 100 * cat + j))
    for t in range(7):
        for j in range(3):
            rows.append(np.full(CH, K2 + t))
            cols.append(20 * k + 5 + j)
            kind.append(np.full(CH, 1000 + 10 * t + j))
    for t in range(48):
        for j in range(6):
            rows.append(np.full(CH, K2 + 7 + t))
            cols.append(20 * k + 8 + j)
            kind.append(np.full(CH, 2000 + 10 * t + j))
    for r in range(8):
        for j in range(3):
            rows.append(np.full(CH, K2 + 55 + r))
            cols.append(20 * k + 14 + j)
            kind.append(np.full(CH, 5000 + 10 * r + j))
    rows = np.concatenate(rows)
    cols = np.concatenate(cols)
    kind = np.concatenate(kind)
    ones = kind == -1
    cat = np.where(ones, 1, kind // 100) % 10 * 0
    vals = jnp.where(
        jnp.asarray(ones), 1.0,
        jnp.where(jnp.asarray(kind >= 5000),
                  fc_W[np.maximum(kind - 5000, 0) % 10,
                       np.maximum(kind - 5000, 0) // 10],
                  jnp.where(jnp.asarray(kind >= 2000),
                            start_time_W[np.maximum(kind - 2000, 0) // 10,
                                         np.maximum(kind - 2000, 0) % 10],
                            jnp.where(jnp.asarray(kind >= 1000),
                                      weekday_W[np.maximum(kind - 1000, 0) // 10,
                                                np.maximum(kind - 1000, 0) % 10],
                                      sem_pt_W[np.maximum(kind, 0) // 100,
                                               np.maximum(kind, 0) % 100]))))
    return jnp.zeros((KS, NL), jnp.float32).at[rows, cols].set(vals)


def _build_S2(weekday_W, start_time_W, fc_W):
    S2 = jnp.zeros((63, 12), jnp.float32)
    S2 = S2.at[0:7, 0:3].set(weekday_W)
    S2 = S2.at[7:55, 3:9].set(start_time_W)
    S2 = S2.at[55:63, 9:12].set(fc_W.T)
    return S2


def _body(lngs_r, lats_r, td_r, spd_r, az_r, spt_r, wde_r, ste_r, semOe_r,
          wd2_r, st2_r, semO2_r, S_r, S2_r, out_r, traj_r):
    spt = spt_r[...]
    pieces = [lngs_r[...], lats_r[...], td_r[...], spd_r[...], az_r[...]]
    pieces += [(spt == kcat).astype(jnp.float32) for kcat in range(1, 9)]
    pieces.append((wde_r[...] == lax.broadcasted_iota(jnp.int32, (PR, 7), 1)
                   ).astype(jnp.float32))
    pieces.append((ste_r[...] == lax.broadcasted_iota(jnp.int32, (PR, 48), 1)
                   ).astype(jnp.float32))
    pieces.append(semOe_r[...])
    X = jnp.concatenate(pieces, axis=1)
    out_r[...] = lax.dot_general(
        X, S_r[...], (((1,), (0,)), ((), ())),
        preferred_element_type=jnp.float32)

    X2 = jnp.concatenate(
        [(wd2_r[...] == lax.broadcasted_iota(jnp.int32, (PB, 7), 1)
          ).astype(jnp.float32),
         (st2_r[...] == lax.broadcasted_iota(jnp.int32, (PB, 48), 1)
          ).astype(jnp.float32),
         semO2_r[...]], axis=1)
    traj_r[...] = lax.dot_general(
        X2, S2_r[...], (((1,), (0,)), ((), ())),
        preferred_element_type=jnp.float32)


@jax.jit
def kernel(weekday, start_time, sem_O, lngs, lats, travel_dis, spd, azimuth,
           sem_pt, weekday_W, start_time_W, sem_pt_W, fc_W):
    wd = weekday.astype(jnp.int32)
    st = start_time.astype(jnp.int32)
    flat = lambda x: x.reshape(R, CH)
    wde = jnp.repeat(wd, G).reshape(R, 1)
    ste = jnp.repeat(st, G).reshape(R, 1)
    semOe = jnp.repeat(sem_O, G, axis=0)
    S = _build_S(weekday_W, start_time_W, sem_pt_W, fc_W)
    S2 = _build_S2(weekday_W, start_time_W, fc_W)

    grid = (R // PR,)
    row = lambda i: (i, 0)
    full = lambda i: (0, 0)
    rblk = lambda w: pl.BlockSpec((PR, w), row)
    bblk = lambda w: pl.BlockSpec((PB, w), row)
    out, traj = pl.pallas_call(
        _body,
        grid=grid,
        in_specs=[
            rblk(CH), rblk(CH), rblk(CH), rblk(CH), rblk(CH), rblk(CH),
            rblk(1), rblk(1), rblk(8),
            bblk(1), bblk(1), bblk(8),
            pl.BlockSpec((KS, NL), full),
            pl.BlockSpec((63, 12), full),
        ],
        out_specs=[
            pl.BlockSpec((PR, NL), row),
            pl.BlockSpec((PB, 12), row),
        ],
        out_shape=[
            jax.ShapeDtypeStruct((R, NL), jnp.float32),
            jax.ShapeDtypeStruct((B, 12), jnp.float32),
        ],
    )(flat(lngs), flat(lats), flat(travel_dis), flat(spd), flat(azimuth),
      flat(sem_pt.astype(jnp.int32)), wde, ste, semOe,
      wd.reshape(B, 1), st.reshape(B, 1), sem_O, S, S2)
    return (out.reshape(B, L, 20), traj)


# speedup vs baseline: 2.4079x; 2.4079x over previous
import jax
import jax.numpy as jnp
from jax import lax
from jax.experimental import pallas as pl

B = 4096
L = 200
PB = 32


def _build_S(sem_pt_W):
    S = jnp.zeros((15, 20), jnp.float32)
    S = S.at[jnp.arange(5), jnp.arange(5)].set(1.0)
    S = S.at[6:15, 17:20].set(sem_pt_W.at[0].set(0.0))
    return S


def _build_S2(weekday_W, start_time_W, fc_W):
    S2 = jnp.zeros((63, 20), jnp.float32)
    S2 = S2.at[0:7, 5:8].set(weekday_W)
    S2 = S2.at[7:55, 8:14].set(start_time_W)
    S2 = S2.at[55:63, 14:17].set(fc_W.T)
    return S2


def _body(lngs_r, lats_r, td_r, spd_r, az_r, spt_r, wd_r, st_r, semO_r,
          S_r, S2_r, out_r, traj_r):
    stacked = jnp.stack(
        [lngs_r[...], lats_r[...], td_r[...], spd_r[...], az_r[...],
         spt_r[...].astype(jnp.float32)], axis=1)
    xb = jnp.transpose(stacked, (0, 2, 1)).reshape(PB * L, 6)
    xoh = (xb[:, 5:6].astype(jnp.int32) ==
           lax.broadcasted_iota(jnp.int32, (PB * L, 9), 1)
           ).astype(jnp.float32)
    xc = jnp.concatenate([xb, xoh], axis=1)
    mm = lax.dot_general(xc, S_r[...], (((1,), (0,)), ((), ())),
                         preferred_element_type=jnp.float32)

    x2 = jnp.concatenate(
        [(wd_r[...] == lax.broadcasted_iota(jnp.int32, (PB, 7), 1)
          ).astype(jnp.float32),
         (st_r[...] == lax.broadcasted_iota(jnp.int32, (PB, 48), 1)
          ).astype(jnp.float32),
         semO_r[...]], axis=1)
    traj20 = lax.dot_general(x2, S2_r[...], (((1,), (0,)), ((), ())),
                             preferred_element_type=jnp.float32)
    out_r[...] = mm.reshape(PB, L, 20) + traj20[:, None, :]
    traj_r[...] = traj20[:, 5:17]


@jax.jit
def kernel(weekday, start_time, sem_O, lngs, lats, travel_dis, spd, azimuth,
           sem_pt, weekday_W, start_time_W, sem_pt_W, fc_W):
    wd2 = weekday.astype(jnp.int32).reshape(B, 1)
    st2 = start_time.astype(jnp.int32).reshape(B, 1)
    S = _build_S(sem_pt_W)
    S2 = _build_S2(weekday_W, start_time_W, fc_W)

    grid = (B // PB,)
    row = lambda i: (i, 0)
    full = lambda i: (0, 0)
    out, traj = pl.pallas_call(
        _body,
        grid=grid,
        in_specs=[
            pl.BlockSpec((PB, L), row),
            pl.BlockSpec((PB, L), row),
            pl.BlockSpec((PB, L), row),
            pl.BlockSpec((PB, L), row),
            pl.BlockSpec((PB, L), row),
            pl.BlockSpec((PB, L), row),
            pl.BlockSpec((PB, 1), row),
            pl.BlockSpec((PB, 1), row),
            pl.BlockSpec((PB, 8), row),
            pl.BlockSpec((15, 20), full),
            pl.BlockSpec((63, 20), full),
        ],
        out_specs=[
            pl.BlockSpec((PB, L, 20), lambda i: (i, 0, 0)),
            pl.BlockSpec((PB, 12), row),
        ],
        out_shape=[
            jax.ShapeDtypeStruct((B, L, 20), jnp.float32),
            jax.ShapeDtypeStruct((B, 12), jnp.float32),
        ],
    )(lngs, lats, travel_dis, spd, azimuth, sem_pt.astype(jnp.int32),
      wd2, st2, sem_O, S, S2)
    return (out, traj)


# speedup vs baseline: 3.1470x; 1.3069x over previous
import jax
import jax.numpy as jnp
from jax import lax
from jax.experimental import pallas as pl

B = 4096
L = 200
PB = 32


def _build_S(sem_pt_W):
    S = jnp.zeros((13, 20), jnp.float32)
    S = S.at[jnp.arange(5), jnp.arange(5)].set(1.0)
    S = S.at[5:13, 17:20].set(sem_pt_W[1:9])
    return S


def _build_S2(weekday_W, start_time_W, fc_W):
    S2 = jnp.zeros((63, 20), jnp.float32)
    S2 = S2.at[0:7, 5:8].set(weekday_W)
    S2 = S2.at[7:55, 8:14].set(start_time_W)
    S2 = S2.at[55:63, 14:17].set(fc_W.T)
    return S2


def _body(lngs_r, lats_r, td_r, spd_r, az_r, spt_r, wd_r, st_r, semO_r,
          S_r, S2_r, out_r, traj_r):
    spt = spt_r[...]
    stacked = jnp.stack(
        [lngs_r[...], lats_r[...], td_r[...], spd_r[...], az_r[...]] +
        [(spt == k).astype(jnp.float32) for k in range(1, 9)],
        axis=1)
    xb = jnp.transpose(stacked, (0, 2, 1)).reshape(PB * L, 13)
    mm = lax.dot_general(xb, S_r[...], (((1,), (0,)), ((), ())),
                         preferred_element_type=jnp.float32)

    x2 = jnp.concatenate(
        [(wd_r[...] == lax.broadcasted_iota(jnp.int32, (PB, 7), 1)
          ).astype(jnp.float32),
         (st_r[...] == lax.broadcasted_iota(jnp.int32, (PB, 48), 1)
          ).astype(jnp.float32),
         semO_r[...]], axis=1)
    traj20 = lax.dot_general(x2, S2_r[...], (((1,), (0,)), ((), ())),
                             preferred_element_type=jnp.float32)
    out_r[...] = mm.reshape(PB, L, 20) + traj20[:, None, :]
    traj_r[...] = traj20[:, 5:17]


@jax.jit
def kernel(weekday, start_time, sem_O, lngs, lats, travel_dis, spd, azimuth,
           sem_pt, weekday_W, start_time_W, sem_pt_W, fc_W):
    wd2 = weekday.astype(jnp.int32).reshape(B, 1)
    st2 = start_time.astype(jnp.int32).reshape(B, 1)
    S = _build_S(sem_pt_W)
    S2 = _build_S2(weekday_W, start_time_W, fc_W)

    grid = (B // PB,)
    row = lambda i: (i, 0)
    full = lambda i: (0, 0)
    out, traj = pl.pallas_call(
        _body,
        grid=grid,
        in_specs=[
            pl.BlockSpec((PB, L), row),
            pl.BlockSpec((PB, L), row),
            pl.BlockSpec((PB, L), row),
            pl.BlockSpec((PB, L), row),
            pl.BlockSpec((PB, L), row),
            pl.BlockSpec((PB, L), row),
            pl.BlockSpec((PB, 1), row),
            pl.BlockSpec((PB, 1), row),
            pl.BlockSpec((PB, 8), row),
            pl.BlockSpec((13, 20), full),
            pl.BlockSpec((63, 20), full),
        ],
        out_specs=[
            pl.BlockSpec((PB, L, 20), lambda i: (i, 0, 0)),
            pl.BlockSpec((PB, 12), row),
        ],
        out_shape=[
            jax.ShapeDtypeStruct((B, L, 20), jnp.float32),
            jax.ShapeDtypeStruct((B, 12), jnp.float32),
        ],
    )(lngs, lats, travel_dis, spd, azimuth, sem_pt.astype(jnp.int32),
      wd2, st2, sem_O, S, S2)
    return (out, traj)
